# baseline (speedup 1.0000x reference)
import functools

import jax
import jax.numpy as jnp
from jax import lax
from jax.experimental import pallas as pl
from jax.experimental.pallas import tpu as pltpu
from jax.experimental.pallas import tpu_sc as plsc

D = 64
R = 16
P = 512

_NC = 2
_NS = 16
_NW = _NC * _NS
_GATHER_CHUNK = 128


def _make_sc_gather(n_idx, emb_dtype):
  assert n_idx % (_NW * _GATHER_CHUNK) == 0
  b_per_w = n_idx // _NW
  n_sub = b_per_w // _GATHER_CHUNK
  mesh = plsc.VectorSubcoreMesh(core_axis_name="c", subcore_axis_name="s")

  @functools.partial(
      pl.kernel,
      mesh=mesh,
      out_type=jax.ShapeDtypeStruct((n_idx, D), emb_dtype),
      scratch_types=[
          pltpu.VMEM((b_per_w,), jnp.int32),
          pltpu.VMEM((b_per_w, D), emb_dtype),
          pltpu.SemaphoreType.DMA,
      ],
      compiler_params=pltpu.CompilerParams(use_tc_tiling_on_sc=False),
  )
  def gather_kernel(table_hbm, idx_hbm, out_hbm, idx_v, rows_v, sem):
    wid = lax.axis_index("s") * _NC + lax.axis_index("c")
    base = wid * b_per_w
    pltpu.sync_copy(idx_hbm.at[pl.ds(base, b_per_w)], idx_v)
    copies = []
    for j in range(n_sub):
      sl = pl.ds(j * _GATHER_CHUNK, _GATHER_CHUNK)
      copies.append(
          pltpu.async_copy(table_hbm.at[idx_v.at[sl]], rows_v.at[sl], sem))
    for c in copies:
      c.wait()
    pltpu.sync_copy(rows_v, out_hbm.at[pl.ds(base, b_per_w)])

  return gather_kernel


def _score_body(lhs_ref, rhs_ref, ridx_ref, diag_ref,
                pos_ref, ln_ref, rn_ref):
  lhs = lhs_ref[0]
  rhs = rhs_ref[0]
  ridx = ridx_ref[0, 0]
  onehot = (ridx[:, None] == lax.broadcasted_iota(jnp.int32, (P, R), 1))
  diag = jnp.dot(onehot.astype(jnp.float32), diag_ref[...],
                 preferred_element_type=jnp.float32)
  rhs_t = rhs * diag
  pos_ref[0] = jnp.sum(lhs * rhs_t, axis=1)[None, :]
  dn = (((1,), (1,)), ((), ()))
  ln_ref[0] = lax.dot_general(rhs_t, lhs, dn,
                              preferred_element_type=jnp.float32)
  rn_ref[0] = lax.dot_general(lhs, rhs_t, dn,
                              preferred_element_type=jnp.float32)


def _make_tc_scores(C):
  return pl.pallas_call(
      _score_body,
      grid=(C,),
      in_specs=[
          pl.BlockSpec((1, P, D), lambda c: (c, 0, 0)),
          pl.BlockSpec((1, P, D), lambda c: (c, 0, 0)),
          pl.BlockSpec((1, 1, P), lambda c: (c, 0, 0)),
          pl.BlockSpec((R, D), lambda c: (0, 0)),
      ],
      out_specs=[
          pl.BlockSpec((1, 1, P), lambda c: (c, 0, 0)),
          pl.BlockSpec((1, P, P), lambda c: (c, 0, 0)),
          pl.BlockSpec((1, P, P), lambda c: (c, 0, 0)),
      ],
      out_shape=[
          jax.ShapeDtypeStruct((C, 1, P), jnp.float32),
          jax.ShapeDtypeStruct((C, P, P), jnp.float32),
          jax.ShapeDtypeStruct((C, P, P), jnp.float32),
      ],
  )


@jax.jit
def kernel(emb, rel_diag, lhs_idx, rhs_idx, rel_idx):
  B = lhs_idx.shape[0]
  C = B // P
  idx_all = jnp.concatenate([lhs_idx, rhs_idx]).astype(jnp.int32)
  gathered = _make_sc_gather(2 * B, emb.dtype)(emb, idx_all)
  lhs_e = gathered[:B].reshape(C, P, D)
  rhs_e = gathered[B:].reshape(C, P, D)
  ridx = rel_idx.astype(jnp.int32).reshape(C, 1, P)
  pos, ln, rn = _make_tc_scores(C)(lhs_e, rhs_e, ridx, rel_diag)
  return pos.reshape(C, P), ln, rn


# speedup vs baseline: 1.2819x; 1.2819x over previous
import functools

import jax
import jax.numpy as jnp
from jax import lax
from jax.experimental import pallas as pl
from jax.experimental.pallas import tpu as pltpu
from jax.experimental.pallas import tpu_sc as plsc

D = 64
R = 16
P = 512

_TW = 2048
_HW = _TW // 2

_NC = 2
_NS = 16
_NW = _NC * _NS
_GCHUNK = 128
_PHASE = 512


def _transpose_body(x_ref, o_ref):
  x = x_ref[...]
  y1 = x[:, :_HW].T
  y2 = x[:, _HW:].T
  o_ref[...] = jnp.concatenate([y1, y2], axis=1)


def _make_pack(n_cols):
  n_blocks = (n_cols + _TW - 1) // _TW
  return pl.pallas_call(
      _transpose_body,
      grid=(n_blocks,),
      in_specs=[pl.BlockSpec((D, _TW), lambda j: (0, j))],
      out_specs=pl.BlockSpec((_HW, 2 * D), lambda j: (j, 0)),
      out_shape=jax.ShapeDtypeStruct((n_blocks * _HW, 2 * D), jnp.float32),
  )


def _make_sc_gather(n_idx, n_rows):
  b_per_w = n_idx // _NW
  mesh = plsc.VectorSubcoreMesh(core_axis_name="c", subcore_axis_name="s")

  @functools.partial(
      pl.kernel,
      mesh=mesh,
      out_type=jax.ShapeDtypeStruct((n_idx, 2 * D), jnp.float32),
      scratch_types=[
          pltpu.VMEM((b_per_w,), jnp.int32),
          pltpu.VMEM((_PHASE, 2 * D), jnp.float32),
          pltpu.SemaphoreType.DMA,
      ],
  )
  def gather_kernel(table_hbm, idx_hbm, out_hbm, idx_v, rows_v, sem):
    wid = lax.axis_index("s") * _NC + lax.axis_index("c")
    base = wid * b_per_w
    pltpu.sync_copy(idx_hbm.at[pl.ds(base, b_per_w)], idx_v)
    for ph in range(b_per_w // _PHASE):
      copies = []
      for j in range(_PHASE // _GCHUNK):
        isl = pl.ds(ph * _PHASE + j * _GCHUNK, _GCHUNK)
        rsl = pl.ds(j * _GCHUNK, _GCHUNK)
        copies.append(
            pltpu.async_copy(table_hbm.at[idx_v.at[isl]], rows_v.at[rsl],
                             sem))
      for c in copies:
        c.wait()
      pltpu.sync_copy(rows_v, out_hbm.at[pl.ds(base + ph * _PHASE, _PHASE)])

  return gather_kernel


def _score_body(lhs_ref, rhs_ref, lidx_ref, ridx_ref, relidx_ref, diag_ref,
                pos_ref, ln_ref, rn_ref):
  gl = lhs_ref[0]
  gr = rhs_ref[0]
  lsel = ((lidx_ref[0, 0] >> 10) & 1)[:, None] == 1
  rsel = ((ridx_ref[0, 0] >> 10) & 1)[:, None] == 1
  lhs = jnp.where(lsel, gl[:, D:], gl[:, :D])
  rhs = jnp.where(rsel, gr[:, D:], gr[:, :D])
  rel = relidx_ref[0, 0]
  onehot = (rel[:, None] == lax.broadcasted_iota(jnp.int32, (P, R), 1))
  diag = jnp.dot(onehot.astype(jnp.float32), diag_ref[...],
                 preferred_element_type=jnp.float32)
  rhs_t = rhs * diag
  pos_ref[0] = jnp.sum(lhs * rhs_t, axis=1)[None, :]
  dn = (((1,), (1,)), ((), ()))
  ln_ref[0] = lax.dot_general(rhs_t, lhs, dn,
                              preferred_element_type=jnp.float32)
  rn_ref[0] = lax.dot_general(lhs, rhs_t, dn,
                              preferred_element_type=jnp.float32)


def _make_tc_scores(C):
  return pl.pallas_call(
      _score_body,
      grid=(C,),
      in_specs=[
          pl.BlockSpec((1, P, 2 * D), lambda c: (c, 0, 0)),
          pl.BlockSpec((1, P, 2 * D), lambda c: (c, 0, 0)),
          pl.BlockSpec((1, 1, P), lambda c: (c, 0, 0)),
          pl.BlockSpec((1, 1, P), lambda c: (c, 0, 0)),
          pl.BlockSpec((1, 1, P), lambda c: (c, 0, 0)),
          pl.BlockSpec((R, D), lambda c: (0, 0)),
      ],
      out_specs=[
          pl.BlockSpec((1, 1, P), lambda c: (c, 0, 0)),
          pl.BlockSpec((1, P, P), lambda c: (c, 0, 0)),
          pl.BlockSpec((1, P, P), lambda c: (c, 0, 0)),
      ],
      out_shape=[
          jax.ShapeDtypeStruct((C, 1, P), jnp.float32),
          jax.ShapeDtypeStruct((C, P, P), jnp.float32),
          jax.ShapeDtypeStruct((C, P, P), jnp.float32),
      ],
  )


@jax.jit
def kernel(emb, rel_diag, lhs_idx, rhs_idx, rel_idx):
  B = lhs_idx.shape[0]
  C = B // P
  V = emb.shape[0]
  packed = _make_pack(V)(emb.T)
  idx_all = jnp.concatenate([lhs_idx, rhs_idx]).astype(jnp.int32)
  gidx = ((idx_all >> 11) << 10) | (idx_all & 1023)
  gathered = _make_sc_gather(2 * B, packed.shape[0])(packed, gidx)
  lhs_g = gathered[:B].reshape(C, P, 2 * D)
  rhs_g = gathered[B:].reshape(C, P, 2 * D)
  lidx = idx_all[:B].reshape(C, 1, P)
  ridx = idx_all[B:].reshape(C, 1, P)
  relidx = rel_idx.astype(jnp.int32).reshape(C, 1, P)
  pos, ln, rn = _make_tc_scores(C)(lhs_g, rhs_g, lidx, ridx, relidx, rel_diag)
  return pos.reshape(C, P), ln, rn


# speedup vs baseline: 1.4545x; 1.1346x over previous
import functools

import jax
import jax.numpy as jnp
from jax import lax
from jax.experimental import pallas as pl
from jax.experimental.pallas import tpu as pltpu
from jax.experimental.pallas import tpu_sc as plsc

D = 64
R = 16
P = 512

_TW = 2048
_HW = _TW // 2

_NC = 2
_NS = 16
_NW = _NC * _NS
_GCHUNK = 128
_PHASE = 512


def _transpose_body(x_ref, o_ref):
  x = x_ref[...]
  z = jnp.concatenate([x[:, :_HW], x[:, _HW:]], axis=0)
  o_ref[...] = z.T


def _make_pack(n_cols):
  n_blocks = (n_cols + _TW - 1) // _TW
  return pl.pallas_call(
      _transpose_body,
      grid=(n_blocks,),
      in_specs=[pl.BlockSpec((D, _TW), lambda j: (0, j))],
      out_specs=pl.BlockSpec((_HW, 2 * D), lambda j: (j, 0)),
      out_shape=jax.ShapeDtypeStruct((n_blocks * _HW, 2 * D), jnp.float32),
      compiler_params=pltpu.CompilerParams(
          dimension_semantics=("parallel",)),
  )


def _make_sc_gather(n_idx, n_rows):
  b_per_w = n_idx // _NW
  mesh = plsc.VectorSubcoreMesh(core_axis_name="c", subcore_axis_name="s")

  @functools.partial(
      pl.kernel,
      mesh=mesh,
      out_type=jax.ShapeDtypeStruct((n_idx, 2 * D), jnp.float32),
      scratch_types=[
          pltpu.VMEM((b_per_w,), jnp.int32),
          pltpu.VMEM((_PHASE, 2 * D), jnp.float32),
          pltpu.SemaphoreType.DMA,
      ],
  )
  def gather_kernel(table_hbm, idx_hbm, out_hbm, idx_v, rows_v, sem):
    wid = lax.axis_index("s") * _NC + lax.axis_index("c")
    base = wid * b_per_w
    pltpu.sync_copy(idx_hbm.at[pl.ds(base, b_per_w)], idx_v)
    for ph in range(b_per_w // _PHASE):
      copies = []
      for j in range(_PHASE // _GCHUNK):
        isl = pl.ds(ph * _PHASE + j * _GCHUNK, _GCHUNK)
        rsl = pl.ds(j * _GCHUNK, _GCHUNK)
        copies.append(
            pltpu.async_copy(table_hbm.at[idx_v.at[isl]], rows_v.at[rsl],
                             sem))
      for c in copies:
        c.wait()
      pltpu.sync_copy(rows_v, out_hbm.at[pl.ds(base + ph * _PHASE, _PHASE)])

  return gather_kernel


def _score_body(lhs_ref, rhs_ref, lidx_ref, ridx_ref, relidx_ref, diag_ref,
                pos_ref, ln_ref, rn_ref):
  gl = lhs_ref[0]
  gr = rhs_ref[0]
  lsel = ((lidx_ref[0, 0] >> 10) & 1)[:, None] == 1
  rsel = ((ridx_ref[0, 0] >> 10) & 1)[:, None] == 1
  lhs = jnp.where(lsel, gl[:, D:], gl[:, :D])
  rhs = jnp.where(rsel, gr[:, D:], gr[:, :D])
  rel = relidx_ref[0, 0]
  onehot = (rel[:, None] == lax.broadcasted_iota(jnp.int32, (P, R), 1))
  diag = jnp.dot(onehot.astype(jnp.float32), diag_ref[...],
                 preferred_element_type=jnp.float32)
  rhs_t = rhs * diag
  pos_ref[0] = jnp.sum(lhs * rhs_t, axis=1)[None, :]
  dn = (((1,), (1,)), ((), ()))
  ln_ref[0] = lax.dot_general(rhs_t, lhs, dn,
                              preferred_element_type=jnp.float32)
  rn_ref[0] = lax.dot_general(lhs, rhs_t, dn,
                              preferred_element_type=jnp.float32)


def _make_tc_scores(C):
  return pl.pallas_call(
      _score_body,
      grid=(C,),
      in_specs=[
          pl.BlockSpec((1, P, 2 * D), lambda c: (c, 0, 0)),
          pl.BlockSpec((1, P, 2 * D), lambda c: (c, 0, 0)),
          pl.BlockSpec((1, 1, P), lambda c: (c, 0, 0)),
          pl.BlockSpec((1, 1, P), lambda c: (c, 0, 0)),
          pl.BlockSpec((1, 1, P), lambda c: (c, 0, 0)),
          pl.BlockSpec((R, D), lambda c: (0, 0)),
      ],
      out_specs=[
          pl.BlockSpec((1, 1, P), lambda c: (c, 0, 0)),
          pl.BlockSpec((1, P, P), lambda c: (c, 0, 0)),
          pl.BlockSpec((1, P, P), lambda c: (c, 0, 0)),
      ],
      out_shape=[
          jax.ShapeDtypeStruct((C, 1, P), jnp.float32),
          jax.ShapeDtypeStruct((C, P, P), jnp.float32),
          jax.ShapeDtypeStruct((C, P, P), jnp.float32),
      ],
      compiler_params=pltpu.CompilerParams(
          dimension_semantics=("parallel",)),
  )


@jax.jit
def kernel(emb, rel_diag, lhs_idx, rhs_idx, rel_idx):
  B = lhs_idx.shape[0]
  C = B // P
  V = emb.shape[0]
  packed = _make_pack(V)(emb.T)
  idx_all = jnp.concatenate([lhs_idx, rhs_idx]).astype(jnp.int32)
  gidx = ((idx_all >> 11) << 10) | (idx_all & 1023)
  gathered = _make_sc_gather(2 * B, packed.shape[0])(packed, gidx)
  lhs_g = gathered[:B].reshape(C, P, 2 * D)
  rhs_g = gathered[B:].reshape(C, P, 2 * D)
  lidx = idx_all[:B].reshape(C, 1, P)
  ridx = idx_all[B:].reshape(C, 1, P)
  relidx = rel_idx.astype(jnp.int32).reshape(C, 1, P)
  pos, ln, rn = _make_tc_scores(C)(lhs_g, rhs_g, lidx, ridx, relidx, rel_diag)
  return pos.reshape(C, P), ln, rn


# speedup vs baseline: 2.4079x; 1.6555x over previous
import functools

import jax
import jax.numpy as jnp
from jax import lax
from jax.experimental import pallas as pl
from jax.experimental.pallas import tpu as pltpu
from jax.experimental.pallas import tpu_sc as plsc

D = 64
R = 16
P = 512

_TW = 8192
_HW = _TW // 2
_LOG_HW = 12

_NC = 2
_NS = 16
_NW = _NC * _NS
_GCHUNK = 128
_PHASE = 512


def _transpose_body(x_ref, o_ref):
  x = x_ref[...]
  z = jnp.concatenate([x[:, :_HW], x[:, _HW:]], axis=0)
  o_ref[...] = z.T


def _make_pack(n_cols):
  n_blocks = (n_cols + _TW - 1) // _TW
  return pl.pallas_call(
      _transpose_body,
      grid=(n_blocks,),
      in_specs=[pl.BlockSpec((D, _TW), lambda j: (0, j))],
      out_specs=pl.BlockSpec((_HW, 2 * D), lambda j: (j, 0)),
      out_shape=jax.ShapeDtypeStruct((n_blocks * _HW, 2 * D), jnp.float32),
      compiler_params=pltpu.CompilerParams(
          dimension_semantics=("parallel",)),
  )


def _make_sc_gather(n_idx, n_rows):
  b_per_w = n_idx // _NW
  mesh = plsc.VectorSubcoreMesh(core_axis_name="c", subcore_axis_name="s")

  @functools.partial(
      pl.kernel,
      mesh=mesh,
      out_type=jax.ShapeDtypeStruct((n_idx, 2 * D), jnp.float32),
      scratch_types=[
          pltpu.VMEM((b_per_w,), jnp.int32),
          pltpu.VMEM((_PHASE, 2 * D), jnp.float32),
          pltpu.SemaphoreType.DMA,
      ],
  )
  def gather_kernel(table_hbm, idx_hbm, out_hbm, idx_v, rows_v, sem):
    wid = lax.axis_index("s") * _NC + lax.axis_index("c")
    base = wid * b_per_w
    pltpu.sync_copy(idx_hbm.at[pl.ds(base, b_per_w)], idx_v)
    for ph in range(b_per_w // _PHASE):
      copies = []
      for j in range(_PHASE // _GCHUNK):
        isl = pl.ds(ph * _PHASE + j * _GCHUNK, _GCHUNK)
        rsl = pl.ds(j * _GCHUNK, _GCHUNK)
        copies.append(
            pltpu.async_copy(table_hbm.at[idx_v.at[isl]], rows_v.at[rsl],
                             sem))
      for c in copies:
        c.wait()
      pltpu.sync_copy(rows_v, out_hbm.at[pl.ds(base + ph * _PHASE, _PHASE)])

  return gather_kernel


def _score_body(lhs_ref, rhs_ref, lidx_ref, ridx_ref, relidx_ref, diag_ref,
                pos_ref, ln_ref, rn_ref):
  gl = lhs_ref[0]
  gr = rhs_ref[0]
  lsel = ((lidx_ref[0, 0] >> _LOG_HW) & 1)[:, None] == 1
  rsel = ((ridx_ref[0, 0] >> _LOG_HW) & 1)[:, None] == 1
  lhs = jnp.where(lsel, gl[:, D:], gl[:, :D])
  rhs = jnp.where(rsel, gr[:, D:], gr[:, :D])
  rel = relidx_ref[0, 0]
  onehot = (rel[:, None] == lax.broadcasted_iota(jnp.int32, (P, R), 1))
  diag = jnp.dot(onehot.astype(jnp.float32), diag_ref[...],
                 preferred_element_type=jnp.float32)
  rhs_t = rhs * diag
  pos_ref[0] = jnp.sum(lhs * rhs_t, axis=1)[None, :]
  dn = (((1,), (1,)), ((), ()))
  ln_ref[0] = lax.dot_general(rhs_t, lhs, dn,
                              preferred_element_type=jnp.float32)
  rn_ref[0] = lax.dot_general(lhs, rhs_t, dn,
                              preferred_element_type=jnp.float32)


def _make_tc_scores(C):
  return pl.pallas_call(
      _score_body,
      grid=(C,),
      in_specs=[
          pl.BlockSpec((1, P, 2 * D), lambda c: (c, 0, 0)),
          pl.BlockSpec((1, P, 2 * D), lambda c: (c, 0, 0)),
          pl.BlockSpec((1, 1, P), lambda c: (c, 0, 0)),
          pl.BlockSpec((1, 1, P), lambda c: (c, 0, 0)),
          pl.BlockSpec((1, 1, P), lambda c: (c, 0, 0)),
          pl.BlockSpec((R, D), lambda c: (0, 0)),
      ],
      out_specs=[
          pl.BlockSpec((1, 1, P), lambda c: (c, 0, 0)),
          pl.BlockSpec((1, P, P), lambda c: (c, 0, 0)),
          pl.BlockSpec((1, P, P), lambda c: (c, 0, 0)),
      ],
      out_shape=[
          jax.ShapeDtypeStruct((C, 1, P), jnp.float32),
          jax.ShapeDtypeStruct((C, P, P), jnp.float32),
          jax.ShapeDtypeStruct((C, P, P), jnp.float32),
      ],
      compiler_params=pltpu.CompilerParams(
          dimension_semantics=("parallel",)),
  )


@jax.jit
def kernel(emb, rel_diag, lhs_idx, rhs_idx, rel_idx):
  B = lhs_idx.shape[0]
  C = B // P
  V = emb.shape[0]
  packed = _make_pack(V)(emb.T)
  idx_all = jnp.concatenate([lhs_idx, rhs_idx]).astype(jnp.int32)
  gidx = (((idx_all >> (_LOG_HW + 1)) << _LOG_HW)
          | (idx_all & (_HW - 1)))
  gathered = _make_sc_gather(2 * B, packed.shape[0])(packed, gidx)
  lhs_g = gathered[:B].reshape(C, P, 2 * D)
  rhs_g = gathered[B:].reshape(C, P, 2 * D)
  lidx = idx_all[:B].reshape(C, 1, P)
  ridx = idx_all[B:].reshape(C, 1, P)
  relidx = rel_idx.astype(jnp.int32).reshape(C, 1, P)
  pos, ln, rn = _make_tc_scores(C)(lhs_g, rhs_g, lidx, ridx, relidx, rel_diag)
  return pos.reshape(C, P), ln, rn
